# baseline (speedup 1.0000x reference)
import jax
import jax.numpy as jnp
from jax.experimental import pallas as pl

N_TOKENS = 32768
INPUT_DIM = 768
N_EXPERTS = 64
BLOCK_T = 4096


def _top2(logits, gp_ref, idx_ref, rows):
    iota = jax.lax.broadcasted_iota(
        jnp.int32, logits.shape, 1).astype(jnp.float32)
    v1 = jnp.max(logits, axis=1, keepdims=True)
    i1 = jnp.min(jnp.where(logits == v1, iota, 64.0), axis=1, keepdims=True)
    masked = jnp.where(iota == i1, -jnp.inf, logits)
    v2 = jnp.max(masked, axis=1, keepdims=True)
    i2 = jnp.min(jnp.where(masked == v2, iota, 64.0), axis=1, keepdims=True)
    p1 = 1.0 / (1.0 + jnp.exp(v2 - v1))
    gp_ref[rows, :] = jnp.concatenate([p1, 1.0 - p1], axis=1)
    idx_ref[rows, :] = jnp.concatenate([i1, i2], axis=1).astype(jnp.int32)


def _gating_body(xa_ref, xb_ref, wt_ref, b_ref, gp_ref, idx_ref):
    half = BLOCK_T // 2
    wt = wt_ref[...]
    bias = b_ref[...]
    la = jnp.dot(xa_ref[...], wt, preferred_element_type=jnp.float32) + bias
    _top2(la, gp_ref, idx_ref, pl.ds(0, half))
    lb = jnp.dot(xb_ref[...], wt, preferred_element_type=jnp.float32) + bias
    _top2(lb, gp_ref, idx_ref, pl.ds(half, half))


def kernel(x, W, b):
    wt = W.T
    b2 = b.reshape(1, N_EXPERTS)
    grid = (N_TOKENS // BLOCK_T,)
    gate_probs, topk_idx = pl.pallas_call(
        _gating_body,
        grid=grid,
        in_specs=[
            pl.BlockSpec((BLOCK_T // 2, INPUT_DIM), lambda i: (2 * i, 0)),
            pl.BlockSpec((BLOCK_T // 2, INPUT_DIM), lambda i: (2 * i + 1, 0)),
            pl.BlockSpec((INPUT_DIM, N_EXPERTS), lambda i: (0, 0)),
            pl.BlockSpec((1, N_EXPERTS), lambda i: (0, 0)),
        ],
        out_specs=[
            pl.BlockSpec((BLOCK_T, 2), lambda i: (i, 0)),
            pl.BlockSpec((BLOCK_T, 2), lambda i: (i, 0)),
        ],
        out_shape=[
            jax.ShapeDtypeStruct((N_TOKENS, 2), jnp.float32),
            jax.ShapeDtypeStruct((N_TOKENS, 2), jnp.int32),
        ],
    )(x, x, wt, b2)
    return (gate_probs, topk_idx)


# speedup vs baseline: 1.0267x; 1.0267x over previous
import jax
import jax.numpy as jnp
from jax.experimental import pallas as pl

N_TOKENS = 32768
INPUT_DIM = 768
N_EXPERTS = 64
BLOCK_T = 4096


def _top2(logits, gp_ref, idx_ref, rows):
    iota = jax.lax.broadcasted_iota(
        jnp.int32, logits.shape, 1).astype(jnp.float32)
    v1 = jnp.max(logits, axis=1, keepdims=True)
    i1 = jnp.min(jnp.where(logits == v1, iota, 64.0), axis=1, keepdims=True)
    masked = jnp.where(iota == i1, -jnp.inf, logits)
    v2 = jnp.max(masked, axis=1, keepdims=True)
    i2 = jnp.min(jnp.where(masked == v2, iota, 64.0), axis=1, keepdims=True)
    p1 = 1.0 / (1.0 + jnp.exp(v2 - v1))
    gp_ref[rows, :] = jnp.concatenate([p1, 1.0 - p1], axis=1)
    idx_ref[rows, :] = jnp.concatenate([i1, i2], axis=1).astype(jnp.int32)


def _gating_body(x_ref, wt_ref, b_ref, gp_ref, idx_ref):
    logits = jnp.dot(x_ref[...], wt_ref[...],
                     preferred_element_type=jnp.float32) + b_ref[...]
    _top2(logits, gp_ref, idx_ref, pl.ds(0, BLOCK_T))


def kernel(x, W, b):
    wt = W.T
    b2 = b.reshape(1, N_EXPERTS)
    grid = (N_TOKENS // BLOCK_T,)
    gate_probs, topk_idx = pl.pallas_call(
        _gating_body,
        grid=grid,
        in_specs=[
            pl.BlockSpec((BLOCK_T, INPUT_DIM), lambda i: (i, 0)),
            pl.BlockSpec((INPUT_DIM, N_EXPERTS), lambda i: (0, 0)),
            pl.BlockSpec((1, N_EXPERTS), lambda i: (0, 0)),
        ],
        out_specs=[
            pl.BlockSpec((BLOCK_T, 2), lambda i: (i, 0)),
            pl.BlockSpec((BLOCK_T, 2), lambda i: (i, 0)),
        ],
        out_shape=[
            jax.ShapeDtypeStruct((N_TOKENS, 2), jnp.float32),
            jax.ShapeDtypeStruct((N_TOKENS, 2), jnp.int32),
        ],
    )(x, wt, b2)
    return (gate_probs, topk_idx)
